# Initial kernel scaffold; baseline (speedup 1.0000x reference)
import jax
import jax.numpy as jnp
from jax.experimental import pallas as pl


def kernel(z, pos, batch, emb, mlp_w1, mlp_b1, mlp_w2, mlp_b2, conv_w1, conv_w2, conv_b2, lin_w, lin_b, lin1_w, lin1_b, lin2_w, lin2_b):
    raise NotImplementedError("write your pallas kernel here")


# speedup vs baseline: 2.6523x; 2.6523x over previous
import functools

import jax
import jax.numpy as jnp
import numpy as np
from jax.experimental import pallas as pl
from jax.experimental.pallas import tpu as pltpu

BLK = 128
GD = 8
LN2 = 0.6931471805599453
CUTOFF = 10.0
CUT2 = CUTOFF * CUTOFF
NSEG = 8
HIGH = jax.lax.Precision.HIGHEST


def _ssp(x):
    return jnp.maximum(x, 0.0) + jnp.log1p(jnp.exp(-jnp.abs(x))) - LN2


def _dot(a, b):
    return jax.lax.dot_general(a, b, (((1,), (0,)), ((), ())),
                               precision=HIGH,
                               preferred_element_type=jnp.float32)


def _embed_body(z_ref, emb_ref, o_ref, *, nz):
    z = z_ref[...]
    oh = (z == jax.lax.broadcasted_iota(jnp.int32, (1, nz), 1)).astype(jnp.float32)
    o_ref[...] = _dot(oh, emb_ref[...])


def _inter_body(coeff_ref, offs_ref, posp_ref, brow_ref, bcol_ref, h_ref,
                w1_ref, b1_ref, w2_ref, b2_ref, cw1_ref, cw2_ref, cb2_ref,
                lw_ref, lb_ref, hout_ref, agg_ref, *, n, hid, ng):
    c = pl.program_id(0)
    coeff = coeff_ref[0, 0]
    offs = offs_ref[...]
    pi_over_cut = np.float32(np.pi) / np.float32(CUTOFF)

    pos_c = posp_ref[pl.ds(c * BLK, BLK), :]
    pos_ct = pos_c.T
    sqc_row = jnp.sum(pos_ct * pos_ct, axis=0, keepdims=True)
    bc_row = brow_ref[:, pl.ds(c * BLK, BLK)]
    idc_row = c * BLK + jax.lax.broadcasted_iota(jnp.int32, (1, BLK), 1)
    bmin = jnp.min(bc_row)
    bmax = jnp.max(bc_row)
    brow_all = brow_ref[...]
    slo = jnp.sum((brow_all < bmin).astype(jnp.int32))
    shi = jnp.sum((brow_all <= bmax).astype(jnp.int32))
    rlo = slo // BLK
    rhi = (shi + BLK - 1) // BLK

    agg_ref[...] = jnp.zeros((BLK, hid), jnp.float32)
    cw1 = cw1_ref[...]
    w1 = w1_ref[...]
    b1 = b1_ref[...]
    w2 = w2_ref[...]
    b2 = b2_ref[...]

    def rbody(r, carry):
        pos_r = posp_ref[pl.ds(r * BLK, BLK), :]
        sqr_col = jnp.sum(pos_r * pos_r, axis=1, keepdims=True)
        gram = _dot(pos_r, pos_ct)
        d2 = sqr_col + sqc_row - 2.0 * gram
        d = jnp.sqrt(jnp.maximum(d2, 0.0) + 1e-12)
        br_col = bcol_ref[pl.ds(r * BLK, BLK), :]
        idr_col = r * BLK + jax.lax.broadcasted_iota(jnp.int32, (BLK, 1), 0)
        m = (br_col == bc_row) & (d2 <= CUT2) & (idr_col != idc_row)
        cc = jnp.where(m, 0.5 * (jnp.cos(d * pi_over_cut) + 1.0), 0.0)
        xx_r = _dot(h_ref[pl.ds(r * BLK, BLK), :], cw1)
        xx8 = jnp.concatenate([xx_r] * GD, axis=0)

        for jb in range(BLK // GD):
            cols = [d[:, jb * GD + j:jb * GD + j + 1] for j in range(GD)]
            ea = jnp.concatenate(
                [jnp.exp(coeff * (col - offs) ** 2) for col in cols], axis=0)
            ccf = jnp.concatenate(
                [cc[:, jb * GD + j:jb * GD + j + 1] for j in range(GD)],
                axis=0)
            t = _ssp(_dot(ea, w1) + b1)
            wf = (_dot(t, w2) + b2) * ccf
            contrib = jnp.sum((wf * xx8).reshape(GD, BLK, hid), axis=1)
            agg_ref[jb * GD:(jb + 1) * GD, :] += contrib
        return carry

    jax.lax.fori_loop(rlo, rhi, rbody, 0, unroll=False)

    xo = _dot(agg_ref[...], cw2_ref[...]) + cb2_ref[...]
    xo = _ssp(xo)
    xo = _dot(xo, lw_ref[...]) + lb_ref[...]
    hout_ref[...] = h_ref[pl.ds(c * BLK, BLK), :] + xo


def _readout_body(brow_ref, h_ref, l1w_ref, l1b_ref, l2w_ref, l2b_ref, o_ref,
                  *, nseg):
    t = _ssp(_dot(h_ref[...], l1w_ref[...]) + l1b_ref[...])
    y = _dot(t, l2w_ref[...]) + l2b_ref[...]
    seg = (brow_ref[...] ==
           jax.lax.broadcasted_iota(jnp.int32, (nseg, 1), 0)).astype(jnp.float32)
    o_ref[...] = _dot(seg, y)


def _full(shape):
    nd = len(shape)
    return pl.BlockSpec(shape, lambda *_c, _nd=nd: (0,) * _nd)


def kernel(z, pos, batch, emb, mlp_w1, mlp_b1, mlp_w2, mlp_b2,
           conv_w1, conv_w2, conv_b2, lin_w, lin_b, lin1_w, lin1_b,
           lin2_w, lin2_b):
    n, _ = pos.shape
    nz, hid = emb.shape
    ni, ng, nf = mlp_w1.shape
    h2 = lin1_w.shape[1]
    out_dim = lin2_w.shape[1]
    nblk = n // BLK

    z2 = z.astype(jnp.int32).reshape(n, 1)
    batch = batch.astype(jnp.int32)
    brow = batch.reshape(1, n)
    bcol = batch.reshape(n, 1)
    posp = jnp.pad(pos.astype(jnp.float32), ((0, 0), (0, 8 - pos.shape[1])))
    offset = jnp.linspace(0.0, CUTOFF, ng)
    coeff = (-0.5 / (offset[1] - offset[0]) ** 2).astype(jnp.float32)
    coeff = coeff.reshape(1, 1)
    offs = offset.astype(jnp.float32).reshape(1, ng)

    h = pl.pallas_call(
        functools.partial(_embed_body, nz=nz),
        grid=(nblk,),
        in_specs=[pl.BlockSpec((BLK, 1), lambda c: (c, 0)), _full((nz, hid))],
        out_specs=pl.BlockSpec((BLK, hid), lambda c: (c, 0)),
        out_shape=jax.ShapeDtypeStruct((n, hid), jnp.float32),
    )(z2, emb)

    inter = pl.pallas_call(
        functools.partial(_inter_body, n=n, hid=hid, ng=ng),
        grid=(nblk,),
        in_specs=[
            _full((1, 1)), _full((1, ng)), _full((n, 8)), _full((1, n)),
            _full((n, 1)), _full((n, hid)), _full((ng, nf)), _full((1, nf)),
            _full((nf, nf)), _full((1, nf)), _full((hid, nf)),
            _full((nf, hid)), _full((1, hid)), _full((hid, hid)),
            _full((1, hid)),
        ],
        out_specs=pl.BlockSpec((BLK, hid), lambda c: (c, 0)),
        out_shape=jax.ShapeDtypeStruct((n, hid), jnp.float32),
        scratch_shapes=[pltpu.VMEM((BLK, hid), jnp.float32)],
    )

    for i in range(ni):
        h = inter(coeff, offs, posp, brow, bcol, h,
                  mlp_w1[i], mlp_b1[i].reshape(1, nf),
                  mlp_w2[i], mlp_b2[i].reshape(1, nf),
                  conv_w1[i], conv_w2[i], conv_b2[i].reshape(1, hid),
                  lin_w[i], lin_b[i].reshape(1, hid))

    out = pl.pallas_call(
        functools.partial(_readout_body, nseg=NSEG),
        in_specs=[_full((1, n)), _full((n, hid)), _full((hid, h2)),
                  _full((1, h2)), _full((h2, out_dim)), _full((1, out_dim))],
        out_specs=_full((NSEG, out_dim)),
        out_shape=jax.ShapeDtypeStruct((NSEG, out_dim), jnp.float32),
    )(brow, h, lin1_w, lin1_b.reshape(1, h2), lin2_w, lin2_b.reshape(1, out_dim))

    return out


# speedup vs baseline: 6.1960x; 2.3361x over previous
import functools

import jax
import jax.numpy as jnp
import numpy as np
from jax.experimental import pallas as pl
from jax.experimental.pallas import tpu as pltpu

BLK = 128
GD = 8
LN2 = 0.6931471805599453
CUTOFF = 10.0
CUT2 = CUTOFF * CUTOFF
NSEG = 8
HIGH = jax.lax.Precision.HIGHEST
FAST = jax.lax.Precision.DEFAULT


def _ssp(x):
    return jnp.maximum(x, 0.0) + jnp.log1p(jnp.exp(-jnp.abs(x))) - LN2


def _dot(a, b, precision=HIGH):
    return jax.lax.dot_general(a, b, (((1,), (0,)), ((), ())),
                               precision=precision,
                               preferred_element_type=jnp.float32)


def _embed_body(z_ref, emb_ref, o_ref, *, nz):
    z = z_ref[...]
    oh = (z == jax.lax.broadcasted_iota(jnp.int32, (1, nz), 1)).astype(jnp.float32)
    o_ref[...] = _dot(oh, emb_ref[...])


def _inter_body(coeff_ref, offs_ref, posp_ref, brow_ref, bcol_ref, h_ref,
                w1_ref, b1_ref, w2_ref, b2_ref, cw1_ref, cw2_ref, cb2_ref,
                lw_ref, lb_ref, hout_ref, agg_ref, *, n, hid, ng):
    c = pl.program_id(0)
    coeff = coeff_ref[0, 0]
    offs = offs_ref[...]
    pi_over_cut = np.float32(np.pi) / np.float32(CUTOFF)

    pos_c = posp_ref[pl.ds(c * BLK, BLK), :]
    pos_ct = pos_c.T
    sqc_row = jnp.sum(pos_ct * pos_ct, axis=0, keepdims=True)
    bc_row = brow_ref[:, pl.ds(c * BLK, BLK)]
    idc_row = c * BLK + jax.lax.broadcasted_iota(jnp.int32, (1, BLK), 1)
    bmin = jnp.min(bc_row)
    bmax = jnp.max(bc_row)
    brow_all = brow_ref[...]
    slo = jnp.sum((brow_all < bmin).astype(jnp.int32))
    shi = jnp.sum((brow_all <= bmax).astype(jnp.int32))
    rlo = slo // BLK
    rhi = (shi + BLK - 1) // BLK

    agg_ref[...] = jnp.zeros((BLK, hid), jnp.float32)
    cw1 = cw1_ref[...]
    w1 = w1_ref[...]
    b1 = b1_ref[...]
    w2 = w2_ref[...]
    b2 = b2_ref[...]

    def rbody(r, carry):
        pos_r = posp_ref[pl.ds(r * BLK, BLK), :]
        sqr_col = jnp.sum(pos_r * pos_r, axis=1, keepdims=True)
        gram = _dot(pos_r, pos_ct)
        d2 = sqr_col + sqc_row - 2.0 * gram
        d = jnp.sqrt(jnp.maximum(d2, 0.0) + 1e-12)
        br_col = bcol_ref[pl.ds(r * BLK, BLK), :]
        idr_col = r * BLK + jax.lax.broadcasted_iota(jnp.int32, (BLK, 1), 0)
        m = (br_col == bc_row) & (d2 <= CUT2) & (idr_col != idc_row)
        cc = jnp.where(m, 0.5 * (jnp.cos(d * pi_over_cut) + 1.0), 0.0)
        xx_r = _dot(h_ref[pl.ds(r * BLK, BLK), :], cw1)
        xx8 = jnp.concatenate([xx_r] * GD, axis=0)

        for jb in range(BLK // GD):
            cols = [d[:, jb * GD + j:jb * GD + j + 1] for j in range(GD)]
            ea = jnp.concatenate(
                [jnp.exp(coeff * (col - offs) ** 2) for col in cols], axis=0)
            ccf = jnp.concatenate(
                [cc[:, jb * GD + j:jb * GD + j + 1] for j in range(GD)],
                axis=0)
            t = _ssp(_dot(ea, w1, FAST) + b1)
            wf = (_dot(t, w2, FAST) + b2) * ccf
            contrib = jnp.sum((wf * xx8).reshape(GD, BLK, hid), axis=1)
            agg_ref[jb * GD:(jb + 1) * GD, :] += contrib
        return carry

    jax.lax.fori_loop(rlo, rhi, rbody, 0, unroll=False)

    xo = _dot(agg_ref[...], cw2_ref[...]) + cb2_ref[...]
    xo = _ssp(xo)
    xo = _dot(xo, lw_ref[...]) + lb_ref[...]
    hout_ref[...] = h_ref[pl.ds(c * BLK, BLK), :] + xo


def _readout_body(brow_ref, h_ref, l1w_ref, l1b_ref, l2w_ref, l2b_ref, o_ref,
                  *, nseg):
    t = _ssp(_dot(h_ref[...], l1w_ref[...]) + l1b_ref[...])
    y = _dot(t, l2w_ref[...]) + l2b_ref[...]
    seg = (brow_ref[...] ==
           jax.lax.broadcasted_iota(jnp.int32, (nseg, 1), 0)).astype(jnp.float32)
    o_ref[...] = _dot(seg, y)


def _full(shape):
    nd = len(shape)
    return pl.BlockSpec(shape, lambda *_c, _nd=nd: (0,) * _nd)


def kernel(z, pos, batch, emb, mlp_w1, mlp_b1, mlp_w2, mlp_b2,
           conv_w1, conv_w2, conv_b2, lin_w, lin_b, lin1_w, lin1_b,
           lin2_w, lin2_b):
    n, _ = pos.shape
    nz, hid = emb.shape
    ni, ng, nf = mlp_w1.shape
    h2 = lin1_w.shape[1]
    out_dim = lin2_w.shape[1]
    nblk = n // BLK

    z2 = z.astype(jnp.int32).reshape(n, 1)
    batch = batch.astype(jnp.int32)
    brow = batch.reshape(1, n)
    bcol = batch.reshape(n, 1)
    posp = jnp.pad(pos.astype(jnp.float32), ((0, 0), (0, 8 - pos.shape[1])))
    offset = jnp.linspace(0.0, CUTOFF, ng)
    coeff = (-0.5 / (offset[1] - offset[0]) ** 2).astype(jnp.float32)
    coeff = coeff.reshape(1, 1)
    offs = offset.astype(jnp.float32).reshape(1, ng)

    h = pl.pallas_call(
        functools.partial(_embed_body, nz=nz),
        grid=(nblk,),
        in_specs=[pl.BlockSpec((BLK, 1), lambda c: (c, 0)), _full((nz, hid))],
        out_specs=pl.BlockSpec((BLK, hid), lambda c: (c, 0)),
        out_shape=jax.ShapeDtypeStruct((n, hid), jnp.float32),
    )(z2, emb)

    inter = pl.pallas_call(
        functools.partial(_inter_body, n=n, hid=hid, ng=ng),
        grid=(nblk,),
        in_specs=[
            _full((1, 1)), _full((1, ng)), _full((n, 8)), _full((1, n)),
            _full((n, 1)), _full((n, hid)), _full((ng, nf)), _full((1, nf)),
            _full((nf, nf)), _full((1, nf)), _full((hid, nf)),
            _full((nf, hid)), _full((1, hid)), _full((hid, hid)),
            _full((1, hid)),
        ],
        out_specs=pl.BlockSpec((BLK, hid), lambda c: (c, 0)),
        out_shape=jax.ShapeDtypeStruct((n, hid), jnp.float32),
        scratch_shapes=[pltpu.VMEM((BLK, hid), jnp.float32)],
    )

    for i in range(ni):
        h = inter(coeff, offs, posp, brow, bcol, h,
                  mlp_w1[i], mlp_b1[i].reshape(1, nf),
                  mlp_w2[i], mlp_b2[i].reshape(1, nf),
                  conv_w1[i], conv_w2[i], conv_b2[i].reshape(1, hid),
                  lin_w[i], lin_b[i].reshape(1, hid))

    out = pl.pallas_call(
        functools.partial(_readout_body, nseg=NSEG),
        in_specs=[_full((1, n)), _full((n, hid)), _full((hid, h2)),
                  _full((1, h2)), _full((h2, out_dim)), _full((1, out_dim))],
        out_specs=_full((NSEG, out_dim)),
        out_shape=jax.ShapeDtypeStruct((NSEG, out_dim), jnp.float32),
    )(brow, h, lin1_w, lin1_b.reshape(1, h2), lin2_w, lin2_b.reshape(1, out_dim))

    return out


# speedup vs baseline: 6.2084x; 1.0020x over previous
import functools

import jax
import jax.numpy as jnp
import numpy as np
from jax.experimental import pallas as pl
from jax.experimental.pallas import tpu as pltpu

BLK = 128
GD = 8
LN2 = 0.6931471805599453
CUTOFF = 10.0
CUT2 = CUTOFF * CUTOFF
NSEG = 8
HIGH = jax.lax.Precision.HIGHEST
FAST = jax.lax.Precision.DEFAULT


def _ssp(x):
    return jnp.maximum(x, 0.0) + jnp.log1p(jnp.exp(-jnp.abs(x))) - LN2


def _dot(a, b, precision=HIGH):
    return jax.lax.dot_general(a, b, (((1,), (0,)), ((), ())),
                               precision=precision,
                               preferred_element_type=jnp.float32)


def _embed_body(z_ref, emb_ref, o_ref, *, nz):
    z = z_ref[...]
    oh = (z == jax.lax.broadcasted_iota(jnp.int32, (1, nz), 1)).astype(jnp.float32)
    o_ref[...] = _dot(oh, emb_ref[...])


def _inter_body(coeff_ref, offs_ref, posp_ref, brow_ref, bcol_ref, h_ref,
                w1_ref, b1_ref, w2_ref, b2_ref, cw1_ref, cw2_ref, cb2_ref,
                lw_ref, lb_ref, hout_ref, agg_ref, *, n, hid, ng):
    c = pl.program_id(0)
    coeff = coeff_ref[0, 0]
    offs = offs_ref[...]
    offsc = offs.T
    pi_over_cut = np.float32(np.pi) / np.float32(CUTOFF)

    pos_c = posp_ref[pl.ds(c * BLK, BLK), :]
    pos_ct = pos_c.T
    sqc_row = jnp.sum(pos_ct * pos_ct, axis=0, keepdims=True)
    bc_row = brow_ref[:, pl.ds(c * BLK, BLK)]
    idc_row = c * BLK + jax.lax.broadcasted_iota(jnp.int32, (1, BLK), 1)
    bmin = jnp.min(bc_row)
    bmax = jnp.max(bc_row)
    brow_all = brow_ref[...]
    slo = jnp.sum((brow_all < bmin).astype(jnp.int32))
    shi = jnp.sum((brow_all <= bmax).astype(jnp.int32))
    rlo = slo // BLK
    rhi = (shi + BLK - 1) // BLK

    agg_ref[...] = jnp.zeros((BLK, hid), jnp.float32)
    cw1 = cw1_ref[...]
    w1 = w1_ref[...]
    b1 = b1_ref[...]
    w2 = w2_ref[...]
    b2 = b2_ref[...]

    def rbody(r, carry):
        pos_r = posp_ref[pl.ds(r * BLK, BLK), :]
        sqr_col = jnp.sum(pos_r * pos_r, axis=1, keepdims=True)
        gram = _dot(pos_r, pos_ct)
        d2 = sqr_col + sqc_row - 2.0 * gram
        d = jnp.sqrt(jnp.maximum(d2, 0.0) + 1e-12)
        br_col = bcol_ref[pl.ds(r * BLK, BLK), :]
        idr_col = r * BLK + jax.lax.broadcasted_iota(jnp.int32, (BLK, 1), 0)
        m = (br_col == bc_row) & (d2 <= CUT2) & (idr_col != idc_row)
        cc = jnp.where(m, 0.5 * (jnp.cos(d * pi_over_cut) + 1.0), 0.0)
        xx_r = _dot(h_ref[pl.ds(r * BLK, BLK), :], cw1)
        xx8 = jnp.concatenate([xx_r] * GD, axis=0)
        dt = d.T

        for jb in range(BLK // GD):
            dr = jnp.concatenate(
                [dt[jb * GD + j:jb * GD + j + 1, :] for j in range(GD)],
                axis=1)
            eat = jnp.exp(coeff * (dr - offsc) ** 2)
            ccf = jnp.concatenate(
                [cc[:, jb * GD + j:jb * GD + j + 1] for j in range(GD)],
                axis=0)
            t = _ssp(jax.lax.dot_general(
                eat, w1, (((0,), (0,)), ((), ())), precision=FAST,
                preferred_element_type=jnp.float32) + b1)
            wf = (_dot(t, w2, FAST) + b2) * ccf
            contrib = jnp.sum((wf * xx8).reshape(GD, BLK, hid), axis=1)
            agg_ref[jb * GD:(jb + 1) * GD, :] += contrib
        return carry

    jax.lax.fori_loop(rlo, rhi, rbody, 0, unroll=False)

    xo = _dot(agg_ref[...], cw2_ref[...]) + cb2_ref[...]
    xo = _ssp(xo)
    xo = _dot(xo, lw_ref[...]) + lb_ref[...]
    hout_ref[...] = h_ref[pl.ds(c * BLK, BLK), :] + xo


def _readout_body(brow_ref, h_ref, l1w_ref, l1b_ref, l2w_ref, l2b_ref, o_ref,
                  *, nseg):
    t = _ssp(_dot(h_ref[...], l1w_ref[...]) + l1b_ref[...])
    y = _dot(t, l2w_ref[...]) + l2b_ref[...]
    seg = (brow_ref[...] ==
           jax.lax.broadcasted_iota(jnp.int32, (nseg, 1), 0)).astype(jnp.float32)
    o_ref[...] = _dot(seg, y)


def _full(shape):
    nd = len(shape)
    return pl.BlockSpec(shape, lambda *_c, _nd=nd: (0,) * _nd)


def kernel(z, pos, batch, emb, mlp_w1, mlp_b1, mlp_w2, mlp_b2,
           conv_w1, conv_w2, conv_b2, lin_w, lin_b, lin1_w, lin1_b,
           lin2_w, lin2_b):
    n, _ = pos.shape
    nz, hid = emb.shape
    ni, ng, nf = mlp_w1.shape
    h2 = lin1_w.shape[1]
    out_dim = lin2_w.shape[1]
    nblk = n // BLK

    z2 = z.astype(jnp.int32).reshape(n, 1)
    batch = batch.astype(jnp.int32)
    brow = batch.reshape(1, n)
    bcol = batch.reshape(n, 1)
    posp = jnp.pad(pos.astype(jnp.float32), ((0, 0), (0, 8 - pos.shape[1])))
    offset = jnp.linspace(0.0, CUTOFF, ng)
    coeff = (-0.5 / (offset[1] - offset[0]) ** 2).astype(jnp.float32)
    coeff = coeff.reshape(1, 1)
    offs = offset.astype(jnp.float32).reshape(1, ng)

    h = pl.pallas_call(
        functools.partial(_embed_body, nz=nz),
        grid=(nblk,),
        in_specs=[pl.BlockSpec((BLK, 1), lambda c: (c, 0)), _full((nz, hid))],
        out_specs=pl.BlockSpec((BLK, hid), lambda c: (c, 0)),
        out_shape=jax.ShapeDtypeStruct((n, hid), jnp.float32),
    )(z2, emb)

    inter = pl.pallas_call(
        functools.partial(_inter_body, n=n, hid=hid, ng=ng),
        grid=(nblk,),
        in_specs=[
            _full((1, 1)), _full((1, ng)), _full((n, 8)), _full((1, n)),
            _full((n, 1)), _full((n, hid)), _full((ng, nf)), _full((1, nf)),
            _full((nf, nf)), _full((1, nf)), _full((hid, nf)),
            _full((nf, hid)), _full((1, hid)), _full((hid, hid)),
            _full((1, hid)),
        ],
        out_specs=pl.BlockSpec((BLK, hid), lambda c: (c, 0)),
        out_shape=jax.ShapeDtypeStruct((n, hid), jnp.float32),
        scratch_shapes=[pltpu.VMEM((BLK, hid), jnp.float32)],
    )

    for i in range(ni):
        h = inter(coeff, offs, posp, brow, bcol, h,
                  mlp_w1[i], mlp_b1[i].reshape(1, nf),
                  mlp_w2[i], mlp_b2[i].reshape(1, nf),
                  conv_w1[i], conv_w2[i], conv_b2[i].reshape(1, hid),
                  lin_w[i], lin_b[i].reshape(1, hid))

    out = pl.pallas_call(
        functools.partial(_readout_body, nseg=NSEG),
        in_specs=[_full((1, n)), _full((n, hid)), _full((hid, h2)),
                  _full((1, h2)), _full((h2, out_dim)), _full((1, out_dim))],
        out_specs=_full((NSEG, out_dim)),
        out_shape=jax.ShapeDtypeStruct((NSEG, out_dim), jnp.float32),
    )(brow, h, lin1_w, lin1_b.reshape(1, h2), lin2_w, lin2_b.reshape(1, out_dim))

    return out
